# Initial kernel scaffold; baseline (speedup 1.0000x reference)
import jax
import jax.numpy as jnp
from jax.experimental import pallas as pl


def kernel(x, table):
    raise NotImplementedError("write your pallas kernel here")


# speedup vs baseline: 3.5612x; 3.5612x over previous
import functools
import math

import jax
import jax.numpy as jnp
from jax import lax
from jax.experimental import pallas as pl
from jax.experimental.pallas import tpu as pltpu
from jax.experimental.pallas import tpu_sc as plsc

_VOCAB = 1000
_D = 64
_B = 16384 * 26
_SCALE = math.sqrt(_D)

_NC = 2
_NS = 16
_NW = _NC * _NS
_BLK = 128
_NBLK = _B // _BLK
_BLK_PER_W = _NBLK // _NW
_NSUB = 8
_NCHUNK = _BLK_PER_W // _NSUB


def _scale_body(t_ref, o_ref):
    o_ref[...] = t_ref[...] * _SCALE


def _scale_table(table):
    return pl.pallas_call(
        _scale_body,
        out_shape=jax.ShapeDtypeStruct(table.shape, table.dtype),
    )(table)


_mesh = plsc.VectorSubcoreMesh(core_axis_name="c", subcore_axis_name="s")


@functools.partial(
    pl.kernel,
    mesh=_mesh,
    out_type=jax.ShapeDtypeStruct((_NBLK, _BLK, _D), jnp.float32),
    scratch_types=[
        pltpu.VMEM((_BLK_PER_W, _BLK), jnp.int32),
        pltpu.VMEM((_NSUB, _BLK, _D), jnp.float32),
        pltpu.SemaphoreType.DMA,
    ],
    compiler_params=pltpu.CompilerParams(use_tc_tiling_on_sc=False),
)
def _gather_kernel(x_hbm, tab_hbm, out_hbm, idx_v, rows_v, sem):
    wid = lax.axis_index("s") * _NC + lax.axis_index("c")
    base = wid * _BLK_PER_W
    pltpu.sync_copy(x_hbm.at[pl.ds(base, _BLK_PER_W)], idx_v)

    def chunk(g, carry):
        copies = []
        for j in range(_NSUB):
            copies.append(
                pltpu.async_copy(
                    tab_hbm.at[idx_v.at[g * _NSUB + j]], rows_v.at[j], sem
                )
            )
        for c in copies:
            c.wait()
        pltpu.sync_copy(rows_v, out_hbm.at[pl.ds(base + g * _NSUB, _NSUB)])
        return carry

    lax.fori_loop(0, _NCHUNK, chunk, 0)


def kernel(x, table):
    scaled = _scale_table(table)
    x3 = x.reshape(_NBLK, _BLK).astype(jnp.int32)
    out = _gather_kernel(x3, scaled)
    return out.reshape(16384, 26, _D)
